# baseline (speedup 1.0000x reference)
import functools
import jax
import jax.numpy as jnp
from jax import lax
from jax.experimental import pallas as pl
from jax.experimental.pallas import tpu as pltpu
from jax.experimental.pallas import tpu_sc as plsc

_N = 10000
_E = 320000
_DIN = 128
_H = 64
_G = 16
_NC = 2
_NS = 16
_CH = 128
_NCH = 160
_NB = 2
_EPT = _CH * _NCH
_EPAD = _EPT * _NS
_NPAD = 10240
_RPT = _NPAD // _NS


def _seg_body(with_cnt, *refs):
    if with_cnt:
        (y_hbm, src_hbm, dst_hbm, agg_hbm, cnt_hbm,
         src_v, dst_v, rowbufs, zbuf, gsems,
         onesbuf, czbuf, acc, cnt_acc) = refs
    else:
        (y_hbm, src_hbm, dst_hbm, agg_hbm,
         src_v, dst_v, rowbufs, zbuf, gsems, acc) = refs

    c = lax.axis_index("c")
    s = lax.axis_index("s")

    pltpu.sync_copy(src_hbm.at[c, s], src_v)
    pltpu.sync_copy(dst_hbm.at[c, s], dst_v)

    z16 = jnp.zeros((16,), jnp.float32)

    @pl.loop(0, 32)
    def _zb(i):
        for k in range(4):
            zbuf[i, pl.ds(k * 16, 16)] = z16

    @pl.loop(0, _RPT // 32)
    def _za(i):
        pltpu.sync_copy(zbuf, acc.at[pl.ds(s * _RPT + i * 32, 32)])

    if with_cnt:
        ones16 = jnp.ones((16,), jnp.float32)

        @pl.loop(0, _CH)
        def _ob(i):
            onesbuf[i, :] = ones16

        @pl.loop(0, 32)
        def _cz(i):
            czbuf[i, :] = z16

        @pl.loop(0, _RPT // 32)
        def _zca(i):
            pltpu.sync_copy(czbuf, cnt_acc.at[pl.ds(s * _RPT + i * 32, 32)])

    plsc.subcore_barrier()

    def _step(b, j, prefetch):
        pltpu.make_async_copy(y_hbm.at[src_v.at[j]], rowbufs[b], gsems[b]).wait()
        pltpu.sync_copy(rowbufs[b], acc.at[dst_v.at[j]], add=True)
        if with_cnt:
            pltpu.sync_copy(onesbuf, cnt_acc.at[dst_v.at[j]], add=True)
        if prefetch:
            pltpu.async_copy(y_hbm.at[src_v.at[j + _NB]], rowbufs[b], gsems[b])

    for b in range(_NB):
        pltpu.async_copy(y_hbm.at[src_v.at[b]], rowbufs[b], gsems[b])

    @pl.loop(0, _NCH // _NB - 1)
    def _main(t):
        for b in range(_NB):
            _step(b, _NB * t + b, True)

    for b in range(_NB):
        _step(b, _NCH - _NB + b, False)

    plsc.subcore_barrier()

    if with_cnt:
        pltpu.sync_copy(cnt_acc.at[pl.ds(s * _RPT, _RPT)],
                        cnt_hbm.at[c, pl.ds(s * _RPT, _RPT)])

    pltpu.sync_copy(acc.at[pl.ds(s * _RPT, _RPT)],
                    agg_hbm.at[c, pl.ds(s * _RPT, _RPT)])


def _make_seg(with_cnt):
    mesh = plsc.VectorSubcoreMesh(core_axis_name="c", subcore_axis_name="s",
                                  num_cores=_NC, num_subcores=_NS)
    out_type = [jax.ShapeDtypeStruct((_NC, _NPAD, _H), jnp.float32)]
    scratch = [
        pltpu.VMEM((_NCH, _CH), jnp.int32),
        pltpu.VMEM((_NCH, _CH), jnp.int32),
        [pltpu.VMEM((_CH, _H), jnp.float32) for _ in range(_NB)],
        pltpu.VMEM((32, 64), jnp.float32),
        [pltpu.SemaphoreType.DMA for _ in range(_NB)],
    ]
    if with_cnt:
        out_type.append(jax.ShapeDtypeStruct((_NC, _NPAD, 16), jnp.float32))
        scratch += [
            pltpu.VMEM((_CH, 16), jnp.float32),
            pltpu.VMEM((32, 16), jnp.float32),
        ]
    scratch += [pltpu.VMEM_SHARED((_NPAD, _H), jnp.float32)]
    if with_cnt:
        scratch += [pltpu.VMEM_SHARED((_NPAD, 16), jnp.float32)]
    return pl.kernel(functools.partial(_seg_body, with_cnt),
                     out_type=out_type, mesh=mesh, scratch_types=scratch,
                     compiler_params=pltpu.CompilerParams(
                         use_tc_tiling_on_sc=False))


@functools.lru_cache(maxsize=None)
def _seg_built(with_cnt):
    return _make_seg(with_cnt)


def _seg_cnt(*args):
    return _seg_built(True)(*args)


def _seg(*args):
    return _seg_built(False)(*args)


def _pre_body(x_ref, w_ref, y_ref):
    for g in range(_NC):
        y_ref[g] = jnp.dot(x_ref[g], w_ref[g], preferred_element_type=jnp.float32)


def _pre(x2, wl1):
    return pl.pallas_call(
        _pre_body,
        out_shape=jax.ShapeDtypeStruct((_NC, _N, _H), jnp.float32),
    )(x2, wl1)


def _ln_relu(h, g, b):
    mu = jnp.mean(h, axis=-1, keepdims=True)
    d = h - mu
    var = jnp.mean(d * d, axis=-1, keepdims=True)
    return jnp.maximum(d * lax.rsqrt(var + 1e-5) * g + b, 0.0)


_BR = 2000


def _post_body(agg_ref, cnt_ref, x_ref, wr_ref, bl_ref, lng_ref, lnb_ref,
               wl_ref, xn_ref, yn_ref):
    mean = agg_ref[0] / jnp.maximum(cnt_ref[0], 1.0)
    h = mean + bl_ref[0] + jnp.dot(x_ref[0], wr_ref[0],
                                   preferred_element_type=jnp.float32)
    xn = _ln_relu(h, lng_ref[0], lnb_ref[0])
    xn_ref[0] = xn
    yn_ref[0] = jnp.dot(xn, wl_ref[0], preferred_element_type=jnp.float32)


def _post(agg2, cnt2, x2, wr, bl, lng, lnb, wl_next):
    din = x2.shape[-1]
    gi = lambda g, i: (g, i, 0)
    g0 = lambda g, i: (g, 0, 0)
    return pl.pallas_call(
        _post_body,
        grid=(_NC, _N // _BR),
        in_specs=[
            pl.BlockSpec((1, _BR, _H), gi),
            pl.BlockSpec((1, _BR, 1), gi),
            pl.BlockSpec((1, _BR, din), gi),
            pl.BlockSpec((1, din, _H), g0),
            pl.BlockSpec((1, 1, _H), g0),
            pl.BlockSpec((1, 1, _H), g0),
            pl.BlockSpec((1, 1, _H), g0),
            pl.BlockSpec((1, _H, _H), g0),
        ],
        out_specs=[pl.BlockSpec((1, _BR, _H), gi),
                   pl.BlockSpec((1, _BR, _H), gi)],
        out_shape=[jax.ShapeDtypeStruct((_NC, _N, _H), jnp.float32),
                   jax.ShapeDtypeStruct((_NC, _N, _H), jnp.float32)],
    )(agg2, cnt2, x2, wr, bl, lng, lnb, wl_next)


def _final_body(x_ref, batch_ref, wk_ref, bk_ref, q_ref, w1_ref, b1_ref,
                w2_ref, b2_ref, logits_ref, fused_ref, attn_ref, za_ref,
                zv_ref):
    zs = []
    for g in range(_NC):
        onehot = (lax.broadcasted_iota(jnp.int32, (_G, _N), 0)
                  == batch_ref[g]).astype(jnp.float32)
        ssum = jnp.dot(onehot, x_ref[g], preferred_element_type=jnp.float32)
        cg = jnp.sum(onehot, axis=1, keepdims=True)
        zs.append(ssum / jnp.maximum(cg, 1.0))
    za, zv = zs
    za_ref[...] = za
    zv_ref[...] = zv
    scale = 1.0 / (float(_H) ** 0.5)
    s_a = jnp.sum((jnp.dot(za, wk_ref[...], preferred_element_type=jnp.float32)
                   + bk_ref[...]) * q_ref[...], axis=-1, keepdims=True) * scale
    s_v = jnp.sum((jnp.dot(zv, wk_ref[...], preferred_element_type=jnp.float32)
                   + bk_ref[...]) * q_ref[...], axis=-1, keepdims=True) * scale
    m = jnp.maximum(s_a, s_v)
    e_a = jnp.exp(s_a - m)
    e_v = jnp.exp(s_v - m)
    den = e_a + e_v
    a_a = e_a / den
    a_v = e_v / den
    attn_ref[...] = jnp.concatenate([a_a, a_v], axis=1)
    fused = a_a * za + a_v * zv
    fused_ref[...] = fused
    h1 = jnp.maximum(jnp.dot(fused, w1_ref[...],
                             preferred_element_type=jnp.float32)
                     + b1_ref[...], 0.0)
    logits_ref[...] = jnp.dot(h1, w2_ref[...],
                              preferred_element_type=jnp.float32) + b2_ref[...]


def _final(x2, batch2, wk, bk, q, w1, b1, w2, b2):
    return pl.pallas_call(
        _final_body,
        out_shape=[jax.ShapeDtypeStruct((_G, 2), jnp.float32),
                   jax.ShapeDtypeStruct((_G, _H), jnp.float32),
                   jax.ShapeDtypeStruct((_G, 2), jnp.float32),
                   jax.ShapeDtypeStruct((_G, _H), jnp.float32),
                   jax.ShapeDtypeStruct((_G, _H), jnp.float32)],
    )(x2, batch2, wk, bk, q, w1, b1, w2, b2)


def _prep_edges(edge_index, src_bias):
    src, dst = edge_index[0], edge_index[1]
    pad = _EPAD - _E
    src_p = jnp.concatenate(
        [src + src_bias, jnp.full((pad,), src_bias, jnp.int32)])
    junk = _N + jnp.arange(pad, dtype=jnp.int32) % (_NPAD - _N)
    dst_p = jnp.concatenate([dst, junk])
    return (src_p.reshape(_NS, _NCH, _CH), dst_p.reshape(_NS, _NCH, _CH))


def _stackp(pa, pv, name, shape):
    return jnp.stack([pa[name], pv[name]]).reshape((_NC,) + shape)


def kernel(artery_x, artery_edge_index, artery_batch,
           vein_x, vein_edge_index, vein_batch, params):
    pa, pv = params['enc_a'], params['enc_v']

    x2 = jnp.stack([artery_x, vein_x])
    batch2 = jnp.stack([artery_batch, vein_batch]).reshape(_NC, 1, _N)

    sa = _prep_edges(artery_edge_index, 0)
    sv = _prep_edges(vein_edge_index, _N)
    src2 = jnp.stack([sa[0], sv[0]])
    dst2 = jnp.stack([sa[1], sv[1]])

    wl = [_stackp(pa[i], pv[i], 'Wl', (pa[i]['Wl'].shape[0], _H))
          for i in range(3)]
    wr = [_stackp(pa[i], pv[i], 'Wr', (pa[i]['Wr'].shape[0], _H))
          for i in range(3)]
    bl = [_stackp(pa[i], pv[i], 'bl', (1, _H)) for i in range(3)]
    lng = [_stackp(pa[i], pv[i], 'ln_g', (1, _H)) for i in range(3)]
    lnb = [_stackp(pa[i], pv[i], 'ln_b', (1, _H)) for i in range(3)]

    y = _pre(x2, wl[0])
    agg, cnt = _seg_cnt(y.reshape(_NC * _N, _H), src2, dst2)
    cnt2 = cnt[:, :, 0:1]
    x2b, y = _post(agg, cnt2, x2, wr[0], bl[0], lng[0], lnb[0], wl[1])
    agg = _seg(y.reshape(_NC * _N, _H), src2, dst2)[0]
    x2c, y = _post(agg, cnt2, x2b, wr[1], bl[1], lng[1], lnb[1], wl[2])
    agg = _seg(y.reshape(_NC * _N, _H), src2, dst2)[0]
    x3, _ = _post(agg, cnt2, x2c, wr[2], bl[2], lng[2], lnb[2], wr[2])
    logits, fused, attn, z_a, z_v = _final(
        x3, batch2,
        params['Wk'], params['bk'].reshape(1, _H), params['q'],
        params['W1'], params['b1'].reshape(1, _H // 2),
        params['W2'], params['b2'].reshape(1, 2))
    return (logits, fused, attn, z_a, z_v)


# speedup vs baseline: 1.6344x; 1.6344x over previous
import functools
import jax
import jax.numpy as jnp
from jax import lax
from jax.experimental import pallas as pl
from jax.experimental.pallas import tpu as pltpu
from jax.experimental.pallas import tpu_sc as plsc

_N = 10000
_E = 320000
_DIN = 128
_H = 64
_G = 16
_NC = 2
_NS = 16
_CH = 128
_NCH = 157
_NB = 2
_EPT = _CH * _NCH
_EPAD = _EPT * _NS
_NPAD = 10240
_RPT = _NPAD // _NS


def _seg_body(with_cnt, *refs):
    if with_cnt:
        (y_hbm, src_hbm, dst_hbm, agg_hbm, cnt_hbm,
         src_v, dst_v, rowbufs, zbuf, gsems,
         onesbuf, czbuf, acc, cnt_acc) = refs
    else:
        (y_hbm, src_hbm, dst_hbm, agg_hbm,
         src_v, dst_v, rowbufs, zbuf, gsems, acc) = refs

    c = lax.axis_index("c")
    s = lax.axis_index("s")

    pltpu.sync_copy(src_hbm.at[c, s], src_v)
    pltpu.sync_copy(dst_hbm.at[c, s], dst_v)

    z16 = jnp.zeros((16,), jnp.float32)

    @pl.loop(0, 32)
    def _zb(i):
        for k in range(4):
            zbuf[i, pl.ds(k * 16, 16)] = z16

    @pl.loop(0, _RPT // 32)
    def _za(i):
        pltpu.sync_copy(zbuf, acc.at[pl.ds(s * _RPT + i * 32, 32)])

    if with_cnt:
        ones16 = jnp.ones((16,), jnp.float32)

        @pl.loop(0, _CH)
        def _ob(i):
            onesbuf[i, :] = ones16

        @pl.loop(0, 32)
        def _cz(i):
            czbuf[i, :] = z16

        @pl.loop(0, _RPT // 32)
        def _zca(i):
            pltpu.sync_copy(czbuf, cnt_acc.at[pl.ds(s * _RPT + i * 32, 32)])

    plsc.subcore_barrier()

    def _scat(buf, j):
        pltpu.sync_copy(buf, acc.at[dst_v.at[j]], add=True)
        if with_cnt:
            pltpu.sync_copy(onesbuf, cnt_acc.at[dst_v.at[j]], add=True)

    pltpu.async_copy(y_hbm.at[src_v.at[0]], rowbufs[0], gsems[0])

    @pl.loop(0, (_NCH - 1) // 2)
    def _main(t):
        j = 2 * t
        pltpu.async_copy(y_hbm.at[src_v.at[j + 1]], rowbufs[1], gsems[1])
        pltpu.make_async_copy(y_hbm.at[src_v.at[j]], rowbufs[0], gsems[0]).wait()
        _scat(rowbufs[0], j)
        pltpu.async_copy(y_hbm.at[src_v.at[j + 2]], rowbufs[0], gsems[0])
        pltpu.make_async_copy(y_hbm.at[src_v.at[j + 1]], rowbufs[1], gsems[1]).wait()
        _scat(rowbufs[1], j + 1)

    pltpu.make_async_copy(y_hbm.at[src_v.at[_NCH - 1]], rowbufs[0], gsems[0]).wait()
    _scat(rowbufs[0], _NCH - 1)

    plsc.subcore_barrier()

    if with_cnt:
        pltpu.sync_copy(cnt_acc.at[pl.ds(s * _RPT, _RPT)],
                        cnt_hbm.at[c, pl.ds(s * _RPT, _RPT)])

    pltpu.sync_copy(acc.at[pl.ds(s * _RPT, _RPT)],
                    agg_hbm.at[c, pl.ds(s * _RPT, _RPT)])


def _make_seg(with_cnt):
    mesh = plsc.VectorSubcoreMesh(core_axis_name="c", subcore_axis_name="s",
                                  num_cores=_NC, num_subcores=_NS)
    out_type = [jax.ShapeDtypeStruct((_NC, _NPAD, _H), jnp.float32)]
    scratch = [
        pltpu.VMEM((_NCH, _CH), jnp.int32),
        pltpu.VMEM((_NCH, _CH), jnp.int32),
        [pltpu.VMEM((_CH, _H), jnp.float32) for _ in range(_NB)],
        pltpu.VMEM((32, 64), jnp.float32),
        [pltpu.SemaphoreType.DMA for _ in range(_NB)],
    ]
    if with_cnt:
        out_type.append(jax.ShapeDtypeStruct((_NC, _NPAD, 16), jnp.float32))
        scratch += [
            pltpu.VMEM((_CH, 16), jnp.float32),
            pltpu.VMEM((32, 16), jnp.float32),
        ]
    scratch += [pltpu.VMEM_SHARED((_NPAD, _H), jnp.float32)]
    if with_cnt:
        scratch += [pltpu.VMEM_SHARED((_NPAD, 16), jnp.float32)]
    return pl.kernel(functools.partial(_seg_body, with_cnt),
                     out_type=out_type, mesh=mesh, scratch_types=scratch,
                     compiler_params=pltpu.CompilerParams(
                         use_tc_tiling_on_sc=False))


@functools.lru_cache(maxsize=None)
def _seg_built(with_cnt):
    return _make_seg(with_cnt)


def _seg_cnt(*args):
    return _seg_built(True)(*args)


def _seg(*args):
    return _seg_built(False)(*args)


def _pre_body(x_ref, w_ref, y_ref):
    for g in range(_NC):
        y_ref[g] = jnp.dot(x_ref[g], w_ref[g], preferred_element_type=jnp.float32)


def _pre(x2, wl1):
    return pl.pallas_call(
        _pre_body,
        out_shape=jax.ShapeDtypeStruct((_NC, _N, _H), jnp.float32),
    )(x2, wl1)


def _ln_relu(h, g, b):
    mu = jnp.mean(h, axis=-1, keepdims=True)
    d = h - mu
    var = jnp.mean(d * d, axis=-1, keepdims=True)
    return jnp.maximum(d * lax.rsqrt(var + 1e-5) * g + b, 0.0)


_BR = 2000


def _post_body(agg_ref, cnt_ref, x_ref, wr_ref, bl_ref, lng_ref, lnb_ref,
               wl_ref, xn_ref, yn_ref):
    mean = agg_ref[0] / jnp.maximum(cnt_ref[0], 1.0)
    h = mean + bl_ref[0] + jnp.dot(x_ref[0], wr_ref[0],
                                   preferred_element_type=jnp.float32)
    xn = _ln_relu(h, lng_ref[0], lnb_ref[0])
    xn_ref[0] = xn
    yn_ref[0] = jnp.dot(xn, wl_ref[0], preferred_element_type=jnp.float32)


def _post(agg2, cnt2, x2, wr, bl, lng, lnb, wl_next):
    din = x2.shape[-1]
    gi = lambda g, i: (g, i, 0)
    g0 = lambda g, i: (g, 0, 0)
    return pl.pallas_call(
        _post_body,
        grid=(_NC, _N // _BR),
        in_specs=[
            pl.BlockSpec((1, _BR, _H), gi),
            pl.BlockSpec((1, _BR, 1), gi),
            pl.BlockSpec((1, _BR, din), gi),
            pl.BlockSpec((1, din, _H), g0),
            pl.BlockSpec((1, 1, _H), g0),
            pl.BlockSpec((1, 1, _H), g0),
            pl.BlockSpec((1, 1, _H), g0),
            pl.BlockSpec((1, _H, _H), g0),
        ],
        out_specs=[pl.BlockSpec((1, _BR, _H), gi),
                   pl.BlockSpec((1, _BR, _H), gi)],
        out_shape=[jax.ShapeDtypeStruct((_NC, _N, _H), jnp.float32),
                   jax.ShapeDtypeStruct((_NC, _N, _H), jnp.float32)],
    )(agg2, cnt2, x2, wr, bl, lng, lnb, wl_next)


def _final_body(x_ref, batch_ref, wk_ref, bk_ref, q_ref, w1_ref, b1_ref,
                w2_ref, b2_ref, logits_ref, fused_ref, attn_ref, za_ref,
                zv_ref):
    zs = []
    for g in range(_NC):
        onehot = (lax.broadcasted_iota(jnp.int32, (_G, _N), 0)
                  == batch_ref[g]).astype(jnp.float32)
        ssum = jnp.dot(onehot, x_ref[g], preferred_element_type=jnp.float32)
        cg = jnp.sum(onehot, axis=1, keepdims=True)
        zs.append(ssum / jnp.maximum(cg, 1.0))
    za, zv = zs
    za_ref[...] = za
    zv_ref[...] = zv
    scale = 1.0 / (float(_H) ** 0.5)
    s_a = jnp.sum((jnp.dot(za, wk_ref[...], preferred_element_type=jnp.float32)
                   + bk_ref[...]) * q_ref[...], axis=-1, keepdims=True) * scale
    s_v = jnp.sum((jnp.dot(zv, wk_ref[...], preferred_element_type=jnp.float32)
                   + bk_ref[...]) * q_ref[...], axis=-1, keepdims=True) * scale
    m = jnp.maximum(s_a, s_v)
    e_a = jnp.exp(s_a - m)
    e_v = jnp.exp(s_v - m)
    den = e_a + e_v
    a_a = e_a / den
    a_v = e_v / den
    attn_ref[...] = jnp.concatenate([a_a, a_v], axis=1)
    fused = a_a * za + a_v * zv
    fused_ref[...] = fused
    h1 = jnp.maximum(jnp.dot(fused, w1_ref[...],
                             preferred_element_type=jnp.float32)
                     + b1_ref[...], 0.0)
    logits_ref[...] = jnp.dot(h1, w2_ref[...],
                              preferred_element_type=jnp.float32) + b2_ref[...]


def _final(x2, batch2, wk, bk, q, w1, b1, w2, b2):
    return pl.pallas_call(
        _final_body,
        out_shape=[jax.ShapeDtypeStruct((_G, 2), jnp.float32),
                   jax.ShapeDtypeStruct((_G, _H), jnp.float32),
                   jax.ShapeDtypeStruct((_G, 2), jnp.float32),
                   jax.ShapeDtypeStruct((_G, _H), jnp.float32),
                   jax.ShapeDtypeStruct((_G, _H), jnp.float32)],
    )(x2, batch2, wk, bk, q, w1, b1, w2, b2)


def _prep_edges(edge_index, src_bias):
    src, dst = edge_index[0], edge_index[1]
    pad = _EPAD - _E
    src_p = jnp.concatenate(
        [src + src_bias, jnp.full((pad,), src_bias, jnp.int32)])
    junk = _N + jnp.arange(pad, dtype=jnp.int32) % (_NPAD - _N)
    dst_p = jnp.concatenate([dst, junk])
    return (src_p.reshape(_NS, _NCH, _CH), dst_p.reshape(_NS, _NCH, _CH))


def _stackp(pa, pv, name, shape):
    return jnp.stack([pa[name], pv[name]]).reshape((_NC,) + shape)


def kernel(artery_x, artery_edge_index, artery_batch,
           vein_x, vein_edge_index, vein_batch, params):
    pa, pv = params['enc_a'], params['enc_v']

    x2 = jnp.stack([artery_x, vein_x])
    batch2 = jnp.stack([artery_batch, vein_batch]).reshape(_NC, 1, _N)

    sa = _prep_edges(artery_edge_index, 0)
    sv = _prep_edges(vein_edge_index, _N)
    src2 = jnp.stack([sa[0], sv[0]])
    dst2 = jnp.stack([sa[1], sv[1]])

    wl = [_stackp(pa[i], pv[i], 'Wl', (pa[i]['Wl'].shape[0], _H))
          for i in range(3)]
    wr = [_stackp(pa[i], pv[i], 'Wr', (pa[i]['Wr'].shape[0], _H))
          for i in range(3)]
    bl = [_stackp(pa[i], pv[i], 'bl', (1, _H)) for i in range(3)]
    lng = [_stackp(pa[i], pv[i], 'ln_g', (1, _H)) for i in range(3)]
    lnb = [_stackp(pa[i], pv[i], 'ln_b', (1, _H)) for i in range(3)]

    y = _pre(x2, wl[0])
    agg, cnt = _seg_cnt(y.reshape(_NC * _N, _H), src2, dst2)
    cnt2 = cnt[:, :, 0:1]
    x2b, y = _post(agg, cnt2, x2, wr[0], bl[0], lng[0], lnb[0], wl[1])
    agg = _seg(y.reshape(_NC * _N, _H), src2, dst2)[0]
    x2c, y = _post(agg, cnt2, x2b, wr[1], bl[1], lng[1], lnb[1], wl[2])
    agg = _seg(y.reshape(_NC * _N, _H), src2, dst2)[0]
    x3, _ = _post(agg, cnt2, x2c, wr[2], bl[2], lng[2], lnb[2], wr[2])
    logits, fused, attn, z_a, z_v = _final(
        x3, batch2,
        params['Wk'], params['bk'].reshape(1, _H), params['q'],
        params['W1'], params['b1'].reshape(1, _H // 2),
        params['W2'], params['b2'].reshape(1, 2))
    return (logits, fused, attn, z_a, z_v)


# speedup vs baseline: 1.6473x; 1.0079x over previous
import functools
import jax
import jax.numpy as jnp
from jax import lax
from jax.experimental import pallas as pl
from jax.experimental.pallas import tpu as pltpu
from jax.experimental.pallas import tpu_sc as plsc

_N = 10000
_E = 320000
_DIN = 128
_H = 64
_G = 16
_NC = 2
_NS = 16
_CH = 128
_NCH = 157
_NB = 2
_EPT = _CH * _NCH
_EPAD = _EPT * _NS
_NPAD = 10240
_RPT = _NPAD // _NS


def _seg_body(with_cnt, *refs):
    if with_cnt:
        (y_hbm, src_hbm, dst_hbm, agg_hbm, cnt_hbm,
         src_v, dst_v, rowbufs, zbuf, gsems,
         onesbuf, czbuf, acc, cnt_acc) = refs
    else:
        (y_hbm, src_hbm, dst_hbm, agg_hbm,
         src_v, dst_v, rowbufs, zbuf, gsems, acc) = refs

    c = lax.axis_index("c")
    s = lax.axis_index("s")

    cp_src = pltpu.async_copy(src_hbm.at[c, s], src_v, gsems[0])
    cp_dst = pltpu.async_copy(dst_hbm.at[c, s], dst_v, gsems[1])

    z16 = jnp.zeros((16,), jnp.float32)

    @pl.loop(0, 32)
    def _zb(i):
        for k in range(4):
            zbuf[i, pl.ds(k * 16, 16)] = z16

    cp_src.wait()
    cp_dst.wait()

    pltpu.async_copy(y_hbm.at[src_v.at[0]], rowbufs[0], gsems[0])

    @pl.loop(0, _RPT // 32)
    def _za(i):
        pltpu.sync_copy(zbuf, acc.at[pl.ds(s * _RPT + i * 32, 32)])

    if with_cnt:
        ones16 = jnp.ones((16,), jnp.float32)

        @pl.loop(0, _CH)
        def _ob(i):
            onesbuf[i, :] = ones16

        @pl.loop(0, 32)
        def _cz(i):
            czbuf[i, :] = z16

        @pl.loop(0, _RPT // 32)
        def _zca(i):
            pltpu.sync_copy(czbuf, cnt_acc.at[pl.ds(s * _RPT + i * 32, 32)])

    plsc.subcore_barrier()

    def _scat(buf, j):
        pltpu.sync_copy(buf, acc.at[dst_v.at[j]], add=True)
        if with_cnt:
            pltpu.sync_copy(onesbuf, cnt_acc.at[dst_v.at[j]], add=True)

    @pl.loop(0, (_NCH - 1) // 2)
    def _main(t):
        j = 2 * t
        pltpu.async_copy(y_hbm.at[src_v.at[j + 1]], rowbufs[1], gsems[1])
        pltpu.make_async_copy(y_hbm.at[src_v.at[j]], rowbufs[0], gsems[0]).wait()
        _scat(rowbufs[0], j)
        pltpu.async_copy(y_hbm.at[src_v.at[j + 2]], rowbufs[0], gsems[0])
        pltpu.make_async_copy(y_hbm.at[src_v.at[j + 1]], rowbufs[1], gsems[1]).wait()
        _scat(rowbufs[1], j + 1)

    pltpu.make_async_copy(y_hbm.at[src_v.at[_NCH - 1]], rowbufs[0], gsems[0]).wait()
    _scat(rowbufs[0], _NCH - 1)

    plsc.subcore_barrier()

    if with_cnt:
        pltpu.sync_copy(cnt_acc.at[pl.ds(s * _RPT, _RPT)],
                        cnt_hbm.at[c, pl.ds(s * _RPT, _RPT)])

    pltpu.sync_copy(acc.at[pl.ds(s * _RPT, _RPT)],
                    agg_hbm.at[c, pl.ds(s * _RPT, _RPT)])


def _make_seg(with_cnt):
    mesh = plsc.VectorSubcoreMesh(core_axis_name="c", subcore_axis_name="s",
                                  num_cores=_NC, num_subcores=_NS)
    out_type = [jax.ShapeDtypeStruct((_NC, _NPAD, _H), jnp.float32)]
    scratch = [
        pltpu.VMEM((_NCH, _CH), jnp.int32),
        pltpu.VMEM((_NCH, _CH), jnp.int32),
        [pltpu.VMEM((_CH, _H), jnp.float32) for _ in range(_NB)],
        pltpu.VMEM((32, 64), jnp.float32),
        [pltpu.SemaphoreType.DMA for _ in range(_NB)],
    ]
    if with_cnt:
        out_type.append(jax.ShapeDtypeStruct((_NC, _NPAD, 16), jnp.float32))
        scratch += [
            pltpu.VMEM((_CH, 16), jnp.float32),
            pltpu.VMEM((32, 16), jnp.float32),
        ]
    scratch += [pltpu.VMEM_SHARED((_NPAD, _H), jnp.float32)]
    if with_cnt:
        scratch += [pltpu.VMEM_SHARED((_NPAD, 16), jnp.float32)]
    return pl.kernel(functools.partial(_seg_body, with_cnt),
                     out_type=out_type, mesh=mesh, scratch_types=scratch,
                     compiler_params=pltpu.CompilerParams(
                         use_tc_tiling_on_sc=False))


@functools.lru_cache(maxsize=None)
def _seg_built(with_cnt):
    return _make_seg(with_cnt)


def _seg_cnt(*args):
    return _seg_built(True)(*args)


def _seg(*args):
    return _seg_built(False)(*args)


def _pre_body(x_ref, w_ref, y_ref):
    for g in range(_NC):
        y_ref[g] = jnp.dot(x_ref[g], w_ref[g], preferred_element_type=jnp.float32)


def _pre(x2, wl1):
    return pl.pallas_call(
        _pre_body,
        out_shape=jax.ShapeDtypeStruct((_NC, _N, _H), jnp.float32),
    )(x2, wl1)


def _ln_relu(h, g, b):
    mu = jnp.mean(h, axis=-1, keepdims=True)
    d = h - mu
    var = jnp.mean(d * d, axis=-1, keepdims=True)
    return jnp.maximum(d * lax.rsqrt(var + 1e-5) * g + b, 0.0)


_BR = 2000


def _post_body(agg_ref, cnt_ref, x_ref, wr_ref, bl_ref, lng_ref, lnb_ref,
               wl_ref, xn_ref, yn_ref):
    mean = agg_ref[0] / jnp.maximum(cnt_ref[0], 1.0)
    h = mean + bl_ref[0] + jnp.dot(x_ref[0], wr_ref[0],
                                   preferred_element_type=jnp.float32)
    xn = _ln_relu(h, lng_ref[0], lnb_ref[0])
    xn_ref[0] = xn
    yn_ref[0] = jnp.dot(xn, wl_ref[0], preferred_element_type=jnp.float32)


def _post(agg2, cnt2, x2, wr, bl, lng, lnb, wl_next):
    din = x2.shape[-1]
    gi = lambda g, i: (g, i, 0)
    g0 = lambda g, i: (g, 0, 0)
    return pl.pallas_call(
        _post_body,
        grid=(_NC, _N // _BR),
        in_specs=[
            pl.BlockSpec((1, _BR, _H), gi),
            pl.BlockSpec((1, _BR, 1), gi),
            pl.BlockSpec((1, _BR, din), gi),
            pl.BlockSpec((1, din, _H), g0),
            pl.BlockSpec((1, 1, _H), g0),
            pl.BlockSpec((1, 1, _H), g0),
            pl.BlockSpec((1, 1, _H), g0),
            pl.BlockSpec((1, _H, _H), g0),
        ],
        out_specs=[pl.BlockSpec((1, _BR, _H), gi),
                   pl.BlockSpec((1, _BR, _H), gi)],
        out_shape=[jax.ShapeDtypeStruct((_NC, _N, _H), jnp.float32),
                   jax.ShapeDtypeStruct((_NC, _N, _H), jnp.float32)],
    )(agg2, cnt2, x2, wr, bl, lng, lnb, wl_next)


def _final_body(x_ref, batch_ref, wk_ref, bk_ref, q_ref, w1_ref, b1_ref,
                w2_ref, b2_ref, logits_ref, fused_ref, attn_ref, za_ref,
                zv_ref):
    zs = []
    for g in range(_NC):
        onehot = (lax.broadcasted_iota(jnp.int32, (_G, _N), 0)
                  == batch_ref[g]).astype(jnp.float32)
        ssum = jnp.dot(onehot, x_ref[g], preferred_element_type=jnp.float32)
        cg = jnp.sum(onehot, axis=1, keepdims=True)
        zs.append(ssum / jnp.maximum(cg, 1.0))
    za, zv = zs
    za_ref[...] = za
    zv_ref[...] = zv
    scale = 1.0 / (float(_H) ** 0.5)
    s_a = jnp.sum((jnp.dot(za, wk_ref[...], preferred_element_type=jnp.float32)
                   + bk_ref[...]) * q_ref[...], axis=-1, keepdims=True) * scale
    s_v = jnp.sum((jnp.dot(zv, wk_ref[...], preferred_element_type=jnp.float32)
                   + bk_ref[...]) * q_ref[...], axis=-1, keepdims=True) * scale
    m = jnp.maximum(s_a, s_v)
    e_a = jnp.exp(s_a - m)
    e_v = jnp.exp(s_v - m)
    den = e_a + e_v
    a_a = e_a / den
    a_v = e_v / den
    attn_ref[...] = jnp.concatenate([a_a, a_v], axis=1)
    fused = a_a * za + a_v * zv
    fused_ref[...] = fused
    h1 = jnp.maximum(jnp.dot(fused, w1_ref[...],
                             preferred_element_type=jnp.float32)
                     + b1_ref[...], 0.0)
    logits_ref[...] = jnp.dot(h1, w2_ref[...],
                              preferred_element_type=jnp.float32) + b2_ref[...]


def _final(x2, batch2, wk, bk, q, w1, b1, w2, b2):
    return pl.pallas_call(
        _final_body,
        out_shape=[jax.ShapeDtypeStruct((_G, 2), jnp.float32),
                   jax.ShapeDtypeStruct((_G, _H), jnp.float32),
                   jax.ShapeDtypeStruct((_G, 2), jnp.float32),
                   jax.ShapeDtypeStruct((_G, _H), jnp.float32),
                   jax.ShapeDtypeStruct((_G, _H), jnp.float32)],
    )(x2, batch2, wk, bk, q, w1, b1, w2, b2)


def _prep_edges(edge_index, src_bias):
    src, dst = edge_index[0], edge_index[1]
    pad = _EPAD - _E
    src_p = jnp.concatenate(
        [src + src_bias, jnp.full((pad,), src_bias, jnp.int32)])
    junk = _N + jnp.arange(pad, dtype=jnp.int32) % (_NPAD - _N)
    dst_p = jnp.concatenate([dst, junk])
    return (src_p.reshape(_NS, _NCH, _CH), dst_p.reshape(_NS, _NCH, _CH))


def _stackp(pa, pv, name, shape):
    return jnp.stack([pa[name], pv[name]]).reshape((_NC,) + shape)


def kernel(artery_x, artery_edge_index, artery_batch,
           vein_x, vein_edge_index, vein_batch, params):
    pa, pv = params['enc_a'], params['enc_v']

    x2 = jnp.stack([artery_x, vein_x])
    batch2 = jnp.stack([artery_batch, vein_batch]).reshape(_NC, 1, _N)

    sa = _prep_edges(artery_edge_index, 0)
    sv = _prep_edges(vein_edge_index, _N)
    src2 = jnp.stack([sa[0], sv[0]])
    dst2 = jnp.stack([sa[1], sv[1]])

    wl = [_stackp(pa[i], pv[i], 'Wl', (pa[i]['Wl'].shape[0], _H))
          for i in range(3)]
    wr = [_stackp(pa[i], pv[i], 'Wr', (pa[i]['Wr'].shape[0], _H))
          for i in range(3)]
    bl = [_stackp(pa[i], pv[i], 'bl', (1, _H)) for i in range(3)]
    lng = [_stackp(pa[i], pv[i], 'ln_g', (1, _H)) for i in range(3)]
    lnb = [_stackp(pa[i], pv[i], 'ln_b', (1, _H)) for i in range(3)]

    y = _pre(x2, wl[0])
    agg, cnt = _seg_cnt(y.reshape(_NC * _N, _H), src2, dst2)
    cnt2 = cnt[:, :, 0:1]
    x2b, y = _post(agg, cnt2, x2, wr[0], bl[0], lng[0], lnb[0], wl[1])
    agg = _seg(y.reshape(_NC * _N, _H), src2, dst2)[0]
    x2c, y = _post(agg, cnt2, x2b, wr[1], bl[1], lng[1], lnb[1], wl[2])
    agg = _seg(y.reshape(_NC * _N, _H), src2, dst2)[0]
    x3, _ = _post(agg, cnt2, x2c, wr[2], bl[2], lng[2], lnb[2], wr[2])
    logits, fused, attn, z_a, z_v = _final(
        x3, batch2,
        params['Wk'], params['bk'].reshape(1, _H), params['q'],
        params['W1'], params['b1'].reshape(1, _H // 2),
        params['W2'], params['b2'].reshape(1, 2))
    return (logits, fused, attn, z_a, z_v)


# speedup vs baseline: 1.6567x; 1.0057x over previous
import functools
import jax
import jax.numpy as jnp
from jax import lax
from jax.experimental import pallas as pl
from jax.experimental.pallas import tpu as pltpu
from jax.experimental.pallas import tpu_sc as plsc

_N = 10000
_E = 320000
_DIN = 128
_H = 64
_G = 16
_NC = 2
_NS = 16
_CH = 128
_NCH = 157
_NB = 2
_EPT = _CH * _NCH
_EPAD = _EPT * _NS
_NPAD = 10240
_RPT = _NPAD // _NS


def _seg_body(with_cnt, *refs):
    if with_cnt:
        (y_hbm, src_hbm, dst_hbm, agg_hbm, cnt_hbm,
         src_v, dst_v, rowbufs, zbuf, gsems,
         onesbuf, czbuf, acc, cnt_acc) = refs
    else:
        (y_hbm, src_hbm, dst_hbm, agg_hbm,
         src_v, dst_v, rowbufs, zbuf, gsems, acc) = refs

    c = lax.axis_index("c")
    s = lax.axis_index("s")

    cp_src = pltpu.async_copy(src_hbm.at[c, s], src_v, gsems[0])
    cp_dst = pltpu.async_copy(dst_hbm.at[c, s], dst_v, gsems[1])

    z16 = jnp.zeros((16,), jnp.float32)

    @pl.loop(0, 32)
    def _zb(i):
        for k in range(4):
            zbuf[i, pl.ds(k * 16, 16)] = z16

    cp_src.wait()
    cp_dst.wait()

    pltpu.async_copy(y_hbm.at[src_v.at[0]], rowbufs[0], gsems[0])

    @pl.loop(0, _RPT // 32)
    def _za(i):
        pltpu.sync_copy(zbuf, acc.at[pl.ds(s * _RPT + i * 32, 32)])

    if with_cnt:
        ones16 = jnp.ones((16,), jnp.float32)

        @pl.loop(0, _CH)
        def _ob(i):
            onesbuf[i, :] = ones16

        @pl.loop(0, 32)
        def _cz(i):
            czbuf[i, :] = z16

        @pl.loop(0, _RPT // 32)
        def _zca(i):
            pltpu.sync_copy(czbuf, cnt_acc.at[pl.ds(s * _RPT + i * 32, 32)])

    plsc.subcore_barrier()

    def _scat(buf, j):
        pltpu.sync_copy(buf, acc.at[dst_v.at[j]], add=True)
        if with_cnt:
            pltpu.sync_copy(onesbuf, cnt_acc.at[dst_v.at[j]], add=True)

    @pl.loop(0, (_NCH - 1) // 2)
    def _main(t):
        j = 2 * t
        pltpu.async_copy(y_hbm.at[src_v.at[j + 1]], rowbufs[1], gsems[1])
        pltpu.make_async_copy(y_hbm.at[src_v.at[j]], rowbufs[0], gsems[0]).wait()
        _scat(rowbufs[0], j)
        pltpu.async_copy(y_hbm.at[src_v.at[j + 2]], rowbufs[0], gsems[0])
        pltpu.make_async_copy(y_hbm.at[src_v.at[j + 1]], rowbufs[1], gsems[1]).wait()
        _scat(rowbufs[1], j + 1)

    pltpu.make_async_copy(y_hbm.at[src_v.at[_NCH - 1]], rowbufs[0], gsems[0]).wait()
    _scat(rowbufs[0], _NCH - 1)

    plsc.subcore_barrier()

    if with_cnt:
        pltpu.sync_copy(cnt_acc.at[pl.ds(s * _RPT, _RPT)],
                        cnt_hbm.at[c, pl.ds(s * _RPT, _RPT)])

    pltpu.sync_copy(acc.at[pl.ds(s * _RPT, _RPT)],
                    agg_hbm.at[c, pl.ds(s * _RPT, _RPT)])


def _make_seg(with_cnt):
    mesh = plsc.VectorSubcoreMesh(core_axis_name="c", subcore_axis_name="s",
                                  num_cores=_NC, num_subcores=_NS)
    out_type = [jax.ShapeDtypeStruct((_NC, _NPAD, _H), jnp.float32)]
    scratch = [
        pltpu.VMEM((_NCH, _CH), jnp.int32),
        pltpu.VMEM((_NCH, _CH), jnp.int32),
        [pltpu.VMEM((_CH, _H), jnp.float32) for _ in range(_NB)],
        pltpu.VMEM((32, 64), jnp.float32),
        [pltpu.SemaphoreType.DMA for _ in range(_NB)],
    ]
    if with_cnt:
        out_type.append(jax.ShapeDtypeStruct((_NC, _NPAD, 16), jnp.float32))
        scratch += [
            pltpu.VMEM((_CH, 16), jnp.float32),
            pltpu.VMEM((32, 16), jnp.float32),
        ]
    scratch += [pltpu.VMEM_SHARED((_NPAD, _H), jnp.float32)]
    if with_cnt:
        scratch += [pltpu.VMEM_SHARED((_NPAD, 16), jnp.float32)]
    return pl.kernel(functools.partial(_seg_body, with_cnt),
                     out_type=out_type, mesh=mesh, scratch_types=scratch,
                     compiler_params=pltpu.CompilerParams(
                         use_tc_tiling_on_sc=False))


@functools.lru_cache(maxsize=None)
def _seg_built(with_cnt):
    return _make_seg(with_cnt)


def _seg_cnt(*args):
    return _seg_built(True)(*args)


def _seg(*args):
    return _seg_built(False)(*args)


def _pre_body(x_ref, w_ref, y_ref):
    for g in range(_NC):
        y_ref[g] = jnp.dot(x_ref[g], w_ref[g], preferred_element_type=jnp.float32)


def _pre(x2, wl1):
    return pl.pallas_call(
        _pre_body,
        out_shape=jax.ShapeDtypeStruct((_NC, _N, _H), jnp.float32),
    )(x2, wl1)


def _ln_relu(h, g, b):
    mu = jnp.mean(h, axis=-1, keepdims=True)
    d = h - mu
    var = jnp.mean(d * d, axis=-1, keepdims=True)
    return jnp.maximum(d * lax.rsqrt(var + 1e-5) * g + b, 0.0)


_BR = 5000


def _post_body(agg_ref, cnt_ref, x_ref, wr_ref, bl_ref, lng_ref, lnb_ref,
               wl_ref, xn_ref, yn_ref):
    mean = agg_ref[0] / jnp.maximum(cnt_ref[0], 1.0)
    h = mean + bl_ref[0] + jnp.dot(x_ref[0], wr_ref[0],
                                   preferred_element_type=jnp.float32)
    xn = _ln_relu(h, lng_ref[0], lnb_ref[0])
    xn_ref[0] = xn
    yn_ref[0] = jnp.dot(xn, wl_ref[0], preferred_element_type=jnp.float32)


def _post(agg2, cnt2, x2, wr, bl, lng, lnb, wl_next):
    din = x2.shape[-1]
    gi = lambda g, i: (g, i, 0)
    g0 = lambda g, i: (g, 0, 0)
    return pl.pallas_call(
        _post_body,
        grid=(_NC, _N // _BR),
        in_specs=[
            pl.BlockSpec((1, _BR, _H), gi),
            pl.BlockSpec((1, _BR, 1), gi),
            pl.BlockSpec((1, _BR, din), gi),
            pl.BlockSpec((1, din, _H), g0),
            pl.BlockSpec((1, 1, _H), g0),
            pl.BlockSpec((1, 1, _H), g0),
            pl.BlockSpec((1, 1, _H), g0),
            pl.BlockSpec((1, _H, _H), g0),
        ],
        out_specs=[pl.BlockSpec((1, _BR, _H), gi),
                   pl.BlockSpec((1, _BR, _H), gi)],
        out_shape=[jax.ShapeDtypeStruct((_NC, _N, _H), jnp.float32),
                   jax.ShapeDtypeStruct((_NC, _N, _H), jnp.float32)],
    )(agg2, cnt2, x2, wr, bl, lng, lnb, wl_next)


def _final_body(x_ref, batch_ref, wk_ref, bk_ref, q_ref, w1_ref, b1_ref,
                w2_ref, b2_ref, logits_ref, fused_ref, attn_ref, za_ref,
                zv_ref):
    zs = []
    for g in range(_NC):
        onehot = (lax.broadcasted_iota(jnp.int32, (_G, _N), 0)
                  == batch_ref[g]).astype(jnp.float32)
        ssum = jnp.dot(onehot, x_ref[g], preferred_element_type=jnp.float32)
        cg = jnp.sum(onehot, axis=1, keepdims=True)
        zs.append(ssum / jnp.maximum(cg, 1.0))
    za, zv = zs
    za_ref[...] = za
    zv_ref[...] = zv
    scale = 1.0 / (float(_H) ** 0.5)
    s_a = jnp.sum((jnp.dot(za, wk_ref[...], preferred_element_type=jnp.float32)
                   + bk_ref[...]) * q_ref[...], axis=-1, keepdims=True) * scale
    s_v = jnp.sum((jnp.dot(zv, wk_ref[...], preferred_element_type=jnp.float32)
                   + bk_ref[...]) * q_ref[...], axis=-1, keepdims=True) * scale
    m = jnp.maximum(s_a, s_v)
    e_a = jnp.exp(s_a - m)
    e_v = jnp.exp(s_v - m)
    den = e_a + e_v
    a_a = e_a / den
    a_v = e_v / den
    attn_ref[...] = jnp.concatenate([a_a, a_v], axis=1)
    fused = a_a * za + a_v * zv
    fused_ref[...] = fused
    h1 = jnp.maximum(jnp.dot(fused, w1_ref[...],
                             preferred_element_type=jnp.float32)
                     + b1_ref[...], 0.0)
    logits_ref[...] = jnp.dot(h1, w2_ref[...],
                              preferred_element_type=jnp.float32) + b2_ref[...]


def _final(x2, batch2, wk, bk, q, w1, b1, w2, b2):
    return pl.pallas_call(
        _final_body,
        out_shape=[jax.ShapeDtypeStruct((_G, 2), jnp.float32),
                   jax.ShapeDtypeStruct((_G, _H), jnp.float32),
                   jax.ShapeDtypeStruct((_G, 2), jnp.float32),
                   jax.ShapeDtypeStruct((_G, _H), jnp.float32),
                   jax.ShapeDtypeStruct((_G, _H), jnp.float32)],
    )(x2, batch2, wk, bk, q, w1, b1, w2, b2)


def _prep_edges(edge_index, src_bias):
    src, dst = edge_index[0], edge_index[1]
    pad = _EPAD - _E
    src_p = jnp.concatenate(
        [src + src_bias, jnp.full((pad,), src_bias, jnp.int32)])
    junk = _N + jnp.arange(pad, dtype=jnp.int32) % (_NPAD - _N)
    dst_p = jnp.concatenate([dst, junk])
    return (src_p.reshape(_NS, _NCH, _CH), dst_p.reshape(_NS, _NCH, _CH))


def _stackp(pa, pv, name, shape):
    return jnp.stack([pa[name], pv[name]]).reshape((_NC,) + shape)


def kernel(artery_x, artery_edge_index, artery_batch,
           vein_x, vein_edge_index, vein_batch, params):
    pa, pv = params['enc_a'], params['enc_v']

    x2 = jnp.stack([artery_x, vein_x])
    batch2 = jnp.stack([artery_batch, vein_batch]).reshape(_NC, 1, _N)

    sa = _prep_edges(artery_edge_index, 0)
    sv = _prep_edges(vein_edge_index, _N)
    src2 = jnp.stack([sa[0], sv[0]])
    dst2 = jnp.stack([sa[1], sv[1]])

    wl = [_stackp(pa[i], pv[i], 'Wl', (pa[i]['Wl'].shape[0], _H))
          for i in range(3)]
    wr = [_stackp(pa[i], pv[i], 'Wr', (pa[i]['Wr'].shape[0], _H))
          for i in range(3)]
    bl = [_stackp(pa[i], pv[i], 'bl', (1, _H)) for i in range(3)]
    lng = [_stackp(pa[i], pv[i], 'ln_g', (1, _H)) for i in range(3)]
    lnb = [_stackp(pa[i], pv[i], 'ln_b', (1, _H)) for i in range(3)]

    y = _pre(x2, wl[0])
    agg, cnt = _seg_cnt(y.reshape(_NC * _N, _H), src2, dst2)
    cnt2 = cnt[:, :, 0:1]
    x2b, y = _post(agg, cnt2, x2, wr[0], bl[0], lng[0], lnb[0], wl[1])
    agg = _seg(y.reshape(_NC * _N, _H), src2, dst2)[0]
    x2c, y = _post(agg, cnt2, x2b, wr[1], bl[1], lng[1], lnb[1], wl[2])
    agg = _seg(y.reshape(_NC * _N, _H), src2, dst2)[0]
    x3, _ = _post(agg, cnt2, x2c, wr[2], bl[2], lng[2], lnb[2], wr[2])
    logits, fused, attn, z_a, z_v = _final(
        x3, batch2,
        params['Wk'], params['bk'].reshape(1, _H), params['q'],
        params['W1'], params['b1'].reshape(1, _H // 2),
        params['W2'], params['b2'].reshape(1, 2))
    return (logits, fused, attn, z_a, z_v)


# speedup vs baseline: 1.6654x; 1.0052x over previous
import functools
import jax
import jax.numpy as jnp
from jax import lax
from jax.experimental import pallas as pl
from jax.experimental.pallas import tpu as pltpu
from jax.experimental.pallas import tpu_sc as plsc

_N = 10000
_E = 320000
_DIN = 128
_H = 64
_G = 16
_NC = 2
_NS = 16
_CH = 128
_NCH = 157
_NB = 2
_EPT = _CH * _NCH
_EPAD = _EPT * _NS
_NPAD = 10240
_RPT = _NPAD // _NS


def _seg_body(with_cnt, *refs):
    if with_cnt:
        (y_hbm, src_hbm, dst_hbm, agg_hbm, cnt_hbm,
         src_v, dst_v, rowbufs, zbuf, gsems, csem,
         onesbuf, czbuf, acc, cnt_acc) = refs
    else:
        (y_hbm, src_hbm, dst_hbm, agg_hbm,
         src_v, dst_v, rowbufs, zbuf, gsems, acc) = refs

    c = lax.axis_index("c")
    s = lax.axis_index("s")

    cp_src = pltpu.async_copy(src_hbm.at[c, s], src_v, gsems[0])
    cp_dst = pltpu.async_copy(dst_hbm.at[c, s], dst_v, gsems[1])

    z16 = jnp.zeros((16,), jnp.float32)

    @pl.loop(0, 32)
    def _zb(i):
        for k in range(4):
            zbuf[i, pl.ds(k * 16, 16)] = z16

    cp_src.wait()
    cp_dst.wait()

    pltpu.async_copy(y_hbm.at[src_v.at[0]], rowbufs[0], gsems[0])

    @pl.loop(0, _RPT // 32)
    def _za(i):
        pltpu.sync_copy(zbuf, acc.at[pl.ds(s * _RPT + i * 32, 32)])

    if with_cnt:
        ones16 = jnp.ones((16,), jnp.float32)

        @pl.loop(0, _CH)
        def _ob(i):
            onesbuf[i, :] = ones16

        @pl.loop(0, 32)
        def _cz(i):
            czbuf[i, :] = z16

        @pl.loop(0, _RPT // 32)
        def _zca(i):
            pltpu.sync_copy(czbuf, cnt_acc.at[pl.ds(s * _RPT + i * 32, 32)])

    plsc.subcore_barrier()

    def _scat(buf, j):
        if with_cnt:
            cp = pltpu.async_copy(onesbuf, cnt_acc.at[dst_v.at[j]], csem,
                                  add=True)
            pltpu.sync_copy(buf, acc.at[dst_v.at[j]], add=True)
            cp.wait()
        else:
            pltpu.sync_copy(buf, acc.at[dst_v.at[j]], add=True)

    @pl.loop(0, (_NCH - 1) // 2)
    def _main(t):
        j = 2 * t
        pltpu.async_copy(y_hbm.at[src_v.at[j + 1]], rowbufs[1], gsems[1])
        pltpu.make_async_copy(y_hbm.at[src_v.at[j]], rowbufs[0], gsems[0]).wait()
        _scat(rowbufs[0], j)
        pltpu.async_copy(y_hbm.at[src_v.at[j + 2]], rowbufs[0], gsems[0])
        pltpu.make_async_copy(y_hbm.at[src_v.at[j + 1]], rowbufs[1], gsems[1]).wait()
        _scat(rowbufs[1], j + 1)

    pltpu.make_async_copy(y_hbm.at[src_v.at[_NCH - 1]], rowbufs[0], gsems[0]).wait()
    _scat(rowbufs[0], _NCH - 1)

    plsc.subcore_barrier()

    if with_cnt:
        pltpu.sync_copy(cnt_acc.at[pl.ds(s * _RPT, _RPT)],
                        cnt_hbm.at[c, pl.ds(s * _RPT, _RPT)])

    pltpu.sync_copy(acc.at[pl.ds(s * _RPT, _RPT)],
                    agg_hbm.at[c, pl.ds(s * _RPT, _RPT)])


def _make_seg(with_cnt):
    mesh = plsc.VectorSubcoreMesh(core_axis_name="c", subcore_axis_name="s",
                                  num_cores=_NC, num_subcores=_NS)
    out_type = [jax.ShapeDtypeStruct((_NC, _NPAD, _H), jnp.float32)]
    scratch = [
        pltpu.VMEM((_NCH, _CH), jnp.int32),
        pltpu.VMEM((_NCH, _CH), jnp.int32),
        [pltpu.VMEM((_CH, _H), jnp.float32) for _ in range(_NB)],
        pltpu.VMEM((32, 64), jnp.float32),
        [pltpu.SemaphoreType.DMA for _ in range(_NB)],
    ]
    if with_cnt:
        out_type.append(jax.ShapeDtypeStruct((_NC, _NPAD, 16), jnp.float32))
        scratch += [
            pltpu.SemaphoreType.DMA,
            pltpu.VMEM((_CH, 16), jnp.float32),
            pltpu.VMEM((32, 16), jnp.float32),
        ]
    scratch += [pltpu.VMEM_SHARED((_NPAD, _H), jnp.float32)]
    if with_cnt:
        scratch += [pltpu.VMEM_SHARED((_NPAD, 16), jnp.float32)]
    return pl.kernel(functools.partial(_seg_body, with_cnt),
                     out_type=out_type, mesh=mesh, scratch_types=scratch,
                     compiler_params=pltpu.CompilerParams(
                         use_tc_tiling_on_sc=False))


@functools.lru_cache(maxsize=None)
def _seg_built(with_cnt):
    return _make_seg(with_cnt)


def _seg_cnt(*args):
    return _seg_built(True)(*args)


def _seg(*args):
    return _seg_built(False)(*args)


def _pre_body(x_ref, w_ref, y_ref):
    for g in range(_NC):
        y_ref[g] = jnp.dot(x_ref[g], w_ref[g], preferred_element_type=jnp.float32)


def _pre(x2, wl1):
    return pl.pallas_call(
        _pre_body,
        out_shape=jax.ShapeDtypeStruct((_NC, _N, _H), jnp.float32),
    )(x2, wl1)


def _ln_relu(h, g, b):
    mu = jnp.mean(h, axis=-1, keepdims=True)
    d = h - mu
    var = jnp.mean(d * d, axis=-1, keepdims=True)
    return jnp.maximum(d * lax.rsqrt(var + 1e-5) * g + b, 0.0)


_BR = 5000


def _post_body(agg_ref, cnt_ref, x_ref, wr_ref, bl_ref, lng_ref, lnb_ref,
               wl_ref, xn_ref, yn_ref):
    mean = agg_ref[0] / jnp.maximum(cnt_ref[0], 1.0)
    h = mean + bl_ref[0] + jnp.dot(x_ref[0], wr_ref[0],
                                   preferred_element_type=jnp.float32)
    xn = _ln_relu(h, lng_ref[0], lnb_ref[0])
    xn_ref[0] = xn
    yn_ref[0] = jnp.dot(xn, wl_ref[0], preferred_element_type=jnp.float32)


def _post(agg2, cnt2, x2, wr, bl, lng, lnb, wl_next):
    din = x2.shape[-1]
    gi = lambda g, i: (g, i, 0)
    g0 = lambda g, i: (g, 0, 0)
    return pl.pallas_call(
        _post_body,
        grid=(_NC, _N // _BR),
        in_specs=[
            pl.BlockSpec((1, _BR, _H), gi),
            pl.BlockSpec((1, _BR, 1), gi),
            pl.BlockSpec((1, _BR, din), gi),
            pl.BlockSpec((1, din, _H), g0),
            pl.BlockSpec((1, 1, _H), g0),
            pl.BlockSpec((1, 1, _H), g0),
            pl.BlockSpec((1, 1, _H), g0),
            pl.BlockSpec((1, _H, _H), g0),
        ],
        out_specs=[pl.BlockSpec((1, _BR, _H), gi),
                   pl.BlockSpec((1, _BR, _H), gi)],
        out_shape=[jax.ShapeDtypeStruct((_NC, _N, _H), jnp.float32),
                   jax.ShapeDtypeStruct((_NC, _N, _H), jnp.float32)],
    )(agg2, cnt2, x2, wr, bl, lng, lnb, wl_next)


def _final_body(x_ref, batch_ref, wk_ref, bk_ref, q_ref, w1_ref, b1_ref,
                w2_ref, b2_ref, logits_ref, fused_ref, attn_ref, za_ref,
                zv_ref):
    zs = []
    for g in range(_NC):
        onehot = (lax.broadcasted_iota(jnp.int32, (_G, _N), 0)
                  == batch_ref[g]).astype(jnp.float32)
        ssum = jnp.dot(onehot, x_ref[g], preferred_element_type=jnp.float32)
        cg = jnp.sum(onehot, axis=1, keepdims=True)
        zs.append(ssum / jnp.maximum(cg, 1.0))
    za, zv = zs
    za_ref[...] = za
    zv_ref[...] = zv
    scale = 1.0 / (float(_H) ** 0.5)
    s_a = jnp.sum((jnp.dot(za, wk_ref[...], preferred_element_type=jnp.float32)
                   + bk_ref[...]) * q_ref[...], axis=-1, keepdims=True) * scale
    s_v = jnp.sum((jnp.dot(zv, wk_ref[...], preferred_element_type=jnp.float32)
                   + bk_ref[...]) * q_ref[...], axis=-1, keepdims=True) * scale
    m = jnp.maximum(s_a, s_v)
    e_a = jnp.exp(s_a - m)
    e_v = jnp.exp(s_v - m)
    den = e_a + e_v
    a_a = e_a / den
    a_v = e_v / den
    attn_ref[...] = jnp.concatenate([a_a, a_v], axis=1)
    fused = a_a * za + a_v * zv
    fused_ref[...] = fused
    h1 = jnp.maximum(jnp.dot(fused, w1_ref[...],
                             preferred_element_type=jnp.float32)
                     + b1_ref[...], 0.0)
    logits_ref[...] = jnp.dot(h1, w2_ref[...],
                              preferred_element_type=jnp.float32) + b2_ref[...]


def _final(x2, batch2, wk, bk, q, w1, b1, w2, b2):
    return pl.pallas_call(
        _final_body,
        out_shape=[jax.ShapeDtypeStruct((_G, 2), jnp.float32),
                   jax.ShapeDtypeStruct((_G, _H), jnp.float32),
                   jax.ShapeDtypeStruct((_G, 2), jnp.float32),
                   jax.ShapeDtypeStruct((_G, _H), jnp.float32),
                   jax.ShapeDtypeStruct((_G, _H), jnp.float32)],
    )(x2, batch2, wk, bk, q, w1, b1, w2, b2)


def _prep_edges(edge_index, src_bias):
    src, dst = edge_index[0], edge_index[1]
    pad = _EPAD - _E
    src_p = jnp.concatenate(
        [src + src_bias, jnp.full((pad,), src_bias, jnp.int32)])
    junk = _N + jnp.arange(pad, dtype=jnp.int32) % (_NPAD - _N)
    dst_p = jnp.concatenate([dst, junk])
    return (src_p.reshape(_NS, _NCH, _CH), dst_p.reshape(_NS, _NCH, _CH))


def _stackp(pa, pv, name, shape):
    return jnp.stack([pa[name], pv[name]]).reshape((_NC,) + shape)


def kernel(artery_x, artery_edge_index, artery_batch,
           vein_x, vein_edge_index, vein_batch, params):
    pa, pv = params['enc_a'], params['enc_v']

    x2 = jnp.stack([artery_x, vein_x])
    batch2 = jnp.stack([artery_batch, vein_batch]).reshape(_NC, 1, _N)

    sa = _prep_edges(artery_edge_index, 0)
    sv = _prep_edges(vein_edge_index, _N)
    src2 = jnp.stack([sa[0], sv[0]])
    dst2 = jnp.stack([sa[1], sv[1]])

    wl = [_stackp(pa[i], pv[i], 'Wl', (pa[i]['Wl'].shape[0], _H))
          for i in range(3)]
    wr = [_stackp(pa[i], pv[i], 'Wr', (pa[i]['Wr'].shape[0], _H))
          for i in range(3)]
    bl = [_stackp(pa[i], pv[i], 'bl', (1, _H)) for i in range(3)]
    lng = [_stackp(pa[i], pv[i], 'ln_g', (1, _H)) for i in range(3)]
    lnb = [_stackp(pa[i], pv[i], 'ln_b', (1, _H)) for i in range(3)]

    y = _pre(x2, wl[0])
    agg, cnt = _seg_cnt(y.reshape(_NC * _N, _H), src2, dst2)
    cnt2 = cnt[:, :, 0:1]
    x2b, y = _post(agg, cnt2, x2, wr[0], bl[0], lng[0], lnb[0], wl[1])
    agg = _seg(y.reshape(_NC * _N, _H), src2, dst2)[0]
    x2c, y = _post(agg, cnt2, x2b, wr[1], bl[1], lng[1], lnb[1], wl[2])
    agg = _seg(y.reshape(_NC * _N, _H), src2, dst2)[0]
    x3, _ = _post(agg, cnt2, x2c, wr[2], bl[2], lng[2], lnb[2], wr[2])
    logits, fused, attn, z_a, z_v = _final(
        x3, batch2,
        params['Wk'], params['bk'].reshape(1, _H), params['q'],
        params['W1'], params['b1'].reshape(1, _H // 2),
        params['W2'], params['b2'].reshape(1, 2))
    return (logits, fused, attn, z_a, z_v)
